# baseline (speedup 1.0000x reference)
import functools
import jax
import jax.numpy as jnp
from jax import lax
from jax.experimental import pallas as pl
from jax.experimental.pallas import tpu as pltpu
from jax.experimental.pallas import tpu_sc as plsc

_W = 128
_NC = 2
_NS = 16


def _sc_body(embed_hbm, out_hbm, idx_v, rows_v, r_sh, sem):
    c = lax.axis_index("c")
    s = lax.axis_index("s")

    lane = lax.broadcasted_iota(jnp.int32, (16,), 0)
    for t in range(4):
        m = s * 64 + t * 16 + lane
        idx = jnp.clip(511 - m, -_W, _W) + _W
        idx_v[pl.ds(t * 16, 16)] = idx
    pltpu.async_copy(embed_hbm.at[idx_v], rows_v, sem).wait()
    pltpu.sync_copy(rows_v, r_sh.at[pl.ds(s * 64, 64)])
    plsc.subcore_barrier()

    w = s * _NC + c
    for k in range(32):
        p = w * 32 + k
        b = p // 512
        i = p % 512
        pltpu.sync_copy(r_sh.at[pl.ds(511 - i, 512)], out_hbm.at[b, i])


def kernel(x, embed):
    bsz, length, _ = x.shape
    d = embed.shape[1]
    mesh = plsc.VectorSubcoreMesh(core_axis_name="c", subcore_axis_name="s")
    run = functools.partial(
        pl.kernel,
        mesh=mesh,
        out_type=jax.ShapeDtypeStruct((bsz, length, length, d), jnp.float32),
        scratch_types=[
            pltpu.VMEM((64,), jnp.int32),
            pltpu.VMEM((64, d), jnp.float32),
            pltpu.VMEM_SHARED((1024, d), jnp.float32),
            pltpu.SemaphoreType.DMA,
        ],
    )(_sc_body)
    return run(embed)


# speedup vs baseline: 1.5955x; 1.5955x over previous
import functools
import jax
import jax.numpy as jnp
from jax import lax
from jax.experimental import pallas as pl
from jax.experimental.pallas import tpu as pltpu
from jax.experimental.pallas import tpu_sc as plsc

_W = 128
_NC = 2
_NS = 16
_BI = 16


def _sc_gather_body(embed_hbm, r_hbm, idx_v, rows_v, sem):
    c = lax.axis_index("c")
    s = lax.axis_index("s")
    w = s * _NC + c

    lane = lax.broadcasted_iota(jnp.int32, (16,), 0)
    for t in range(2):
        m = w * 32 + t * 16 + lane
        idx = jnp.clip(511 - m, -_W, _W) + _W
        idx_v[pl.ds(t * 16, 16)] = idx
    pltpu.async_copy(embed_hbm.at[idx_v], rows_v, sem).wait()
    pltpu.sync_copy(rows_v, r_hbm.at[pl.ds(w * 32, 32)])


def _tc_broadcast_body(r_ref, out_ref):
    ib = pl.program_id(1)
    for ii in range(_BI):
        i = ib * _BI + ii
        out_ref[0, ii] = r_ref[pl.ds(511 - i, 512), :]


def kernel(x, embed):
    bsz, length, _ = x.shape
    d = embed.shape[1]

    mesh = plsc.VectorSubcoreMesh(core_axis_name="c", subcore_axis_name="s")
    sc_gather = functools.partial(
        pl.kernel,
        mesh=mesh,
        out_type=jax.ShapeDtypeStruct((1024, d), jnp.float32),
        scratch_types=[
            pltpu.VMEM((32,), jnp.int32),
            pltpu.VMEM((32, d), jnp.float32),
            pltpu.SemaphoreType.DMA,
        ],
    )(_sc_gather_body)
    r = sc_gather(embed)

    return pl.pallas_call(
        _tc_broadcast_body,
        grid=(bsz, length // _BI),
        in_specs=[pl.BlockSpec((1024, d), lambda bb, ib: (0, 0))],
        out_specs=pl.BlockSpec((1, _BI, length, d), lambda bb, ib: (bb, ib, 0, 0)),
        out_shape=jax.ShapeDtypeStruct((bsz, length, length, d), jnp.float32),
    )(r)


# speedup vs baseline: 1.6130x; 1.0110x over previous
import functools
import jax
import jax.numpy as jnp
from jax import lax
from jax.experimental import pallas as pl
from jax.experimental.pallas import tpu as pltpu
from jax.experimental.pallas import tpu_sc as plsc

_W = 128
_NC = 2
_NS = 16
_BI = 16


def _sc_gather_body(embed_hbm, r_hbm, idx_v, rows_v, sem):
    c = lax.axis_index("c")
    s = lax.axis_index("s")
    w = s + _NS * c

    lane = lax.broadcasted_iota(jnp.int32, (16,), 0)
    for t in range(4):
        m = w * 64 + t * 16 + lane
        idx = jnp.clip(511 - m, -_W, _W) + _W
        idx_v[pl.ds(t * 16, 16)] = idx
    pltpu.async_copy(embed_hbm.at[idx_v], rows_v, sem).wait()
    pltpu.sync_copy(rows_v, r_hbm.at[pl.ds(w * 64, 64)])


def _tc_broadcast_body(r_ref, out_ref):
    ib = pl.program_id(1)
    for ii in range(_BI):
        i = ib * _BI + ii
        out_ref[0, ii] = r_ref[pl.ds(511 - i, 512), :]


def kernel(x, embed):
    bsz, length, _ = x.shape
    d = embed.shape[1]

    mesh = plsc.VectorSubcoreMesh(
        core_axis_name="c", subcore_axis_name="s", num_cores=1
    )
    sc_gather = functools.partial(
        pl.kernel,
        mesh=mesh,
        out_type=jax.ShapeDtypeStruct((1024, d), jnp.float32),
        scratch_types=[
            pltpu.VMEM((64,), jnp.int32),
            pltpu.VMEM((64, d), jnp.float32),
            pltpu.SemaphoreType.DMA,
        ],
    )(_sc_gather_body)
    r = sc_gather(embed)

    return pl.pallas_call(
        _tc_broadcast_body,
        grid=(bsz, length // _BI),
        in_specs=[pl.BlockSpec((1024, d), lambda bb, ib: (0, 0))],
        out_specs=pl.BlockSpec((1, _BI, length, d), lambda bb, ib: (bb, ib, 0, 0)),
        out_shape=jax.ShapeDtypeStruct((bsz, length, length, d), jnp.float32),
    )(r)
